# Initial kernel scaffold; baseline (speedup 1.0000x reference)
import jax
import jax.numpy as jnp
from jax.experimental import pallas as pl


def kernel(x):
    raise NotImplementedError("write your pallas kernel here")


# speedup vs baseline: 1.1651x; 1.1651x over previous
import functools

import jax
import jax.numpy as jnp
from jax import lax
from jax.experimental import pallas as pl
from jax.experimental.pallas import tpu as pltpu
from jax.experimental.pallas import tpu_sc as plsc

B = 32
C_IN = 384
C_OUT = 768
HW = 32 * 32
XWORDS = C_IN * HW
OWORDS = C_OUT * HW
NW = 32

CHUNK = 32768
NCHUNK = XWORDS // CHUNK
ZWORDS = 16384
NZ = XWORDS // ZWORDS

_MESH = plsc.VectorSubcoreMesh(core_axis_name="c", subcore_axis_name="s")


@functools.partial(
    pl.kernel,
    mesh=_MESH,
    out_type=jax.ShapeDtypeStruct((B * OWORDS,), jnp.float32),
    scratch_types=[
        pltpu.VMEM((ZWORDS,), jnp.float32),
        pltpu.VMEM((CHUNK,), jnp.float32),
        pltpu.VMEM((CHUNK,), jnp.float32),
        pltpu.SemaphoreType.DMA,
        pltpu.SemaphoreType.DMA,
        pltpu.SemaphoreType.DMA,
        pltpu.SemaphoreType.DMA,
        pltpu.SemaphoreType.DMA,
    ],
)
def _sc_pad(x_hbm, z_hbm, out_hbm, zbuf, buf0, buf1,
            zsem, lsem0, lsem1, ssem0, ssem1):
    wid = lax.axis_index("s") * 2 + lax.axis_index("c")
    xbase = wid * XWORDS
    obase = wid * OWORDS

    bufs = (buf0, buf1)
    lsems = (lsem0, lsem1)
    ssems = (ssem0, ssem1)

    zstage = pltpu.async_copy(z_hbm, zbuf, zsem)

    loads = [None, None]
    stores = [None, None]
    loads[0] = pltpu.async_copy(
        x_hbm.at[pl.ds(xbase, CHUNK)], buf0, lsems[0])

    zstage.wait()
    zstores = []
    for k in range(NZ):
        zstores.append(pltpu.async_copy(
            zbuf, out_hbm.at[pl.ds(obase + XWORDS + k * ZWORDS, ZWORDS)],
            zsem))

    for k in range(NCHUNK):
        i = k % 2
        j = (k + 1) % 2
        if k + 1 < NCHUNK:
            if k >= 1:
                stores[j].wait()
            loads[j] = pltpu.async_copy(
                x_hbm.at[pl.ds(xbase + (k + 1) * CHUNK, CHUNK)],
                bufs[j], lsems[j])
        loads[i].wait()
        stores[i] = pltpu.async_copy(
            bufs[i], out_hbm.at[pl.ds(obase + k * CHUNK, CHUNK)], ssems[i])

    stores[0].wait()
    stores[1].wait()
    for h in zstores:
        h.wait()


def kernel(x):
    xf = x.reshape(B * XWORDS)
    z = jnp.zeros((ZWORDS,), jnp.float32)
    outf = _sc_pad(xf, z)
    return outf.reshape(B, C_OUT, 32, 32)


# speedup vs baseline: 9.8750x; 8.4759x over previous
import functools

import jax
import jax.numpy as jnp
from jax import lax
from jax.experimental import pallas as pl
from jax.experimental.pallas import tpu as pltpu
from jax.experimental.pallas import tpu_sc as plsc

B = 32
C_IN = 384
C_OUT = 768
H = 32
W = 32
R = B * H * W
RPB = H * W
CH = 64
NCH = RPB // CH

_MESH = plsc.VectorSubcoreMesh(core_axis_name="c", subcore_axis_name="s")


@functools.partial(
    pl.kernel,
    mesh=_MESH,
    out_type=jax.ShapeDtypeStruct((R, C_OUT), jnp.float32),
    scratch_types=[
        pltpu.VMEM((CH, C_IN), jnp.float32),
        pltpu.VMEM((CH, C_IN), jnp.float32),
        pltpu.VMEM((CH, C_IN), jnp.float32),
        pltpu.SemaphoreType.DMA,
        pltpu.SemaphoreType.DMA,
        pltpu.SemaphoreType.DMA,
        pltpu.SemaphoreType.DMA,
        pltpu.SemaphoreType.DMA,
    ],
)
def _sc_pad(x_hbm, z_hbm, out_hbm, zbuf, buf0, buf1,
            zsem, lsem0, lsem1, ssem0, ssem1):
    wid = lax.axis_index("s") * 2 + lax.axis_index("c")
    r0 = wid * RPB

    bufs = (buf0, buf1)
    lsems = (lsem0, lsem1)
    ssems = (ssem0, ssem1)

    zstage = pltpu.async_copy(z_hbm, zbuf, zsem)

    loads = [None, None]
    stores = [None, None]
    loads[0] = pltpu.async_copy(x_hbm.at[pl.ds(r0, CH)], buf0, lsems[0])

    zstage.wait()
    zstores = []
    for k in range(NCH):
        zstores.append(pltpu.async_copy(
            zbuf,
            out_hbm.at[pl.ds(r0 + k * CH, CH), pl.ds(C_IN, C_IN)],
            zsem))

    for k in range(NCH):
        i = k % 2
        j = (k + 1) % 2
        if k + 1 < NCH:
            if k >= 1:
                stores[j].wait()
            loads[j] = pltpu.async_copy(
                x_hbm.at[pl.ds(r0 + (k + 1) * CH, CH)], bufs[j], lsems[j])
        loads[i].wait()
        stores[i] = pltpu.async_copy(
            bufs[i],
            out_hbm.at[pl.ds(r0 + k * CH, CH), pl.ds(0, C_IN)],
            ssems[i])

    stores[0].wait()
    stores[1].wait()
    for h in zstores:
        h.wait()


def kernel(x):
    x2 = x.transpose(0, 2, 3, 1).reshape(R, C_IN)
    z = jnp.zeros((CH, C_IN), jnp.float32)
    out2 = _sc_pad(x2, z)
    return out2.reshape(B, H, W, C_OUT).transpose(0, 3, 1, 2)


# speedup vs baseline: 10.0878x; 1.0216x over previous
import functools

import jax
import jax.numpy as jnp
from jax import lax
from jax.experimental import pallas as pl
from jax.experimental.pallas import tpu as pltpu
from jax.experimental.pallas import tpu_sc as plsc

B = 32
C_IN = 384
C_OUT = 768
H = 32
W = 32
R = B * H * W
RPB = H * W
CH = 128
NCH = RPB // CH
ZCH = 64
NZ = RPB // ZCH

_MESH = plsc.VectorSubcoreMesh(core_axis_name="c", subcore_axis_name="s")


@functools.partial(
    pl.kernel,
    mesh=_MESH,
    out_type=jax.ShapeDtypeStruct((R, C_OUT), jnp.float32),
    scratch_types=[
        pltpu.VMEM((ZCH, C_IN), jnp.float32),
        pltpu.VMEM((CH, C_IN), jnp.float32),
        pltpu.VMEM((CH, C_IN), jnp.float32),
        pltpu.SemaphoreType.DMA,
        pltpu.SemaphoreType.DMA,
        pltpu.SemaphoreType.DMA,
        pltpu.SemaphoreType.DMA,
        pltpu.SemaphoreType.DMA,
    ],
)
def _sc_pad(x_hbm, z_hbm, out_hbm, zbuf, buf0, buf1,
            zsem, lsem0, lsem1, ssem0, ssem1):
    wid = lax.axis_index("s") * 2 + lax.axis_index("c")
    r0 = wid * RPB

    bufs = (buf0, buf1)
    lsems = (lsem0, lsem1)
    ssems = (ssem0, ssem1)

    zstage = pltpu.async_copy(z_hbm, zbuf, zsem)

    loads = [None, None]
    stores = [None, None]
    loads[0] = pltpu.async_copy(x_hbm.at[pl.ds(r0, CH)], buf0, lsems[0])

    zstage.wait()
    zstores = []
    for k in range(NZ):
        zstores.append(pltpu.async_copy(
            zbuf,
            out_hbm.at[pl.ds(r0 + k * ZCH, ZCH), pl.ds(C_IN, C_IN)],
            zsem))

    for k in range(NCH):
        i = k % 2
        j = (k + 1) % 2
        if k + 1 < NCH:
            if k >= 1:
                stores[j].wait()
            loads[j] = pltpu.async_copy(
                x_hbm.at[pl.ds(r0 + (k + 1) * CH, CH)], bufs[j], lsems[j])
        loads[i].wait()
        stores[i] = pltpu.async_copy(
            bufs[i],
            out_hbm.at[pl.ds(r0 + k * CH, CH), pl.ds(0, C_IN)],
            ssems[i])

    stores[0].wait()
    stores[1].wait()
    for h in zstores:
        h.wait()


def kernel(x):
    x2 = x.transpose(0, 2, 3, 1).reshape(R, C_IN)
    z = jnp.zeros((ZCH, C_IN), jnp.float32)
    out2 = _sc_pad(x2, z)
    return out2.reshape(B, H, W, C_OUT).transpose(0, 3, 1, 2)


# speedup vs baseline: 11.2345x; 1.1137x over previous
import functools

import jax
import jax.numpy as jnp
from jax import lax
from jax.experimental import pallas as pl
from jax.experimental.pallas import tpu as pltpu
from jax.experimental.pallas import tpu_sc as plsc
from jax._src.pallas import mpmd as plmpmd

B = 32
C_IN = 384
C_OUT = 768
H = 32
W = 32
R = B * H * W
RPB = H * W
CH = 128
NCH = RPB // CH
ZR = 256
NZB = RPB // ZR
NCS = 2
BPC = B // NCS

_SCS_MESH = plsc.ScalarSubcoreMesh(axis_name="c")
_TEC_MESH = plsc.VectorSubcoreMesh(core_axis_name="c", subcore_axis_name="s")


def _scs_fn(x_hbm, z_hbm, out_hbm, zsp, zsem,
            buf0, buf1, lsem0, lsem1, ssem0, ssem1):
    del x_hbm, buf0, buf1, lsem0, lsem1, ssem0, ssem1
    c = lax.axis_index("c")

    pltpu.async_copy(z_hbm, zsp, zsem).wait()
    stores = []
    for k in range(BPC):
        r0 = (c * BPC + k) * RPB
        for j in range(NZB):
            stores.append(pltpu.async_copy(
                zsp,
                out_hbm.at[pl.ds(r0 + j * ZR, ZR), pl.ds(C_IN, C_IN)],
                zsem))
    for h in stores:
        h.wait()


def _tec_fn(x_hbm, z_hbm, out_hbm, zsp, zsem,
            buf0, buf1, lsem0, lsem1, ssem0, ssem1):
    del z_hbm, zsp, zsem
    wid = lax.axis_index("c") * 16 + lax.axis_index("s")
    r0 = wid * RPB

    bufs = (buf0, buf1)
    lsems = (lsem0, lsem1)
    ssems = (ssem0, ssem1)
    loads = [None, None]
    stores = [None, None]
    loads[0] = pltpu.async_copy(x_hbm.at[pl.ds(r0, CH)], buf0, lsems[0])
    for k in range(NCH):
        i = k % 2
        j = (k + 1) % 2
        if k + 1 < NCH:
            if k >= 1:
                stores[j].wait()
            loads[j] = pltpu.async_copy(
                x_hbm.at[pl.ds(r0 + (k + 1) * CH, CH)], bufs[j], lsems[j])
        loads[i].wait()
        stores[i] = pltpu.async_copy(
            bufs[i],
            out_hbm.at[pl.ds(r0 + k * CH, CH), pl.ds(0, C_IN)],
            ssems[i])
    stores[0].wait()
    stores[1].wait()


def _sc_pad(x2, z):
    outs = plmpmd.mpmd_map(
        [(_SCS_MESH, _scs_fn), (_TEC_MESH, _tec_fn)],
        out_types=[jax.ShapeDtypeStruct((R, C_OUT), jnp.float32)],
        scratch_types=[
            pltpu.VMEM_SHARED((ZR, C_IN), jnp.float32),
            pltpu.SemaphoreType.DMA @ _SCS_MESH,
            (pltpu.MemorySpace.VMEM @ _TEC_MESH)((CH, C_IN), jnp.float32),
            (pltpu.MemorySpace.VMEM @ _TEC_MESH)((CH, C_IN), jnp.float32),
            pltpu.SemaphoreType.DMA @ _TEC_MESH,
            pltpu.SemaphoreType.DMA @ _TEC_MESH,
            pltpu.SemaphoreType.DMA @ _TEC_MESH,
            pltpu.SemaphoreType.DMA @ _TEC_MESH,
        ],
    )(x2, z)
    return outs[0]


def kernel(x):
    x2 = x.transpose(0, 2, 3, 1).reshape(R, C_IN)
    z = jnp.zeros((ZR, C_IN), jnp.float32)
    out2 = _sc_pad(x2, z)
    return out2.reshape(B, H, W, C_OUT).transpose(0, 3, 1, 2)
